# baseline (speedup 1.0000x reference)
import functools
import math

import jax
import jax.numpy as jnp
import numpy as np
from jax import lax
from jax.experimental import pallas as pl
from jax.experimental.pallas import tpu as pltpu

BS = 50
NUM_NODES = 1000
N = BS * NUM_NODES
E = 800000
HID = 64
EMB = 32
SIGMA = 25.0

EBLK = 2000


def _mm_kernel(r_ref, w_ref, o_ref):
    o_ref[...] = jax.lax.dot_general(
        r_ref[...], w_ref[...], (((1,), (0,)), ((), ())),
        preferred_element_type=jnp.float32)


def _edge_matmul(r, w):
    e, k = r.shape
    d = w.shape[1]
    grid = (e // EBLK,)
    return pl.pallas_call(
        _mm_kernel,
        grid=grid,
        in_specs=[
            pl.BlockSpec((EBLK, k), lambda i: (i, 0)),
            pl.BlockSpec((k, d), lambda i: (0, 0)),
        ],
        out_specs=pl.BlockSpec((EBLK, d), lambda i: (i, 0)),
        out_shape=jax.ShapeDtypeStruct((e, d), jnp.float32),
    )(r, w)


def _layer(feat, src, dst, W1, b1, W2, b2):
    d_in = feat.shape[1]
    W1a = W1[:, :d_in]
    W1b = W1[:, d_in:]
    A = feat @ (W1a - W1b).T + b1
    B = feat @ W1b.T
    R = jax.nn.relu(A[dst] + B[src])
    H = _edge_matmul(R, W2.T)
    agg = jax.ops.segment_max(H, dst, num_segments=N)
    nonempty = jnp.isfinite(agg[:, :1])
    out = jnp.where(nonempty, agg + b2, 0.0)
    return out


def kernel(x, edge_index, batch, t, init_lin_W, init_lin_b, fourier_W,
           embed_W, embed_b, m1_W1, m1_b1, m1_W2, m1_b2, m2_W1, m2_b1,
           m2_W2, m2_b2, m3_W1, m3_b1, m3_W2, m3_b2):
    src = edge_index[0]
    dst = edge_index[1]
    order = jnp.argsort(dst)
    src = src[order]
    dst = dst[order]
    init_feat = jax.nn.relu(x @ init_lin_W.T + init_lin_b)
    ts = t[:, 0]
    proj = ts[:, None] * fourier_W[None, :] * 2.0 * np.pi
    four = jnp.concatenate([jnp.sin(proj), jnp.cos(proj)], axis=-1)
    emb = jax.nn.relu(four @ embed_W.T + embed_b)
    x_sigma = jnp.repeat(emb, NUM_NODES, axis=0)

    h = jax.nn.relu(_layer(init_feat, src, dst, m1_W1, m1_b1, m1_W2, m1_b2))
    h = jnp.concatenate([h, x_sigma], axis=-1)
    h = jax.nn.relu(_layer(h, src, dst, m2_W1, m2_b1, m2_W2, m2_b2))
    h = jnp.concatenate([h, x_sigma], axis=-1)
    out = _layer(h, src, dst, m3_W1, m3_b1, m3_W2, m3_b2)
    std = jnp.sqrt((SIGMA ** (2.0 * jnp.repeat(ts, NUM_NODES)[:, None]) - 1.0)
                   / (2.0 * jnp.log(SIGMA)))
    return out / (std + 1e-07)


# speedup vs baseline: 1.9922x; 1.9922x over previous
import functools

import jax
import jax.numpy as jnp
import numpy as np
from jax import lax
from jax.experimental import pallas as pl
from jax.experimental.pallas import tpu as pltpu
from jax.experimental.pallas import tpu_sc as plsc

BS = 50
NUM_NODES = 1000
N = BS * NUM_NODES
E = 800000
HID = 64
EMB = 32
SIGMA = 25.0

NTILES = 32
NB = 64
NT = 800
NPAD = NB * NT
SUB = 128
CG = SUB
EPT = 196 * CG
E2 = NTILES * EPT
CS = 128
EBLK = 2048


def _vmesh():
    return plsc.VectorSubcoreMesh(core_axis_name="c", subcore_axis_name="s")


def _sc_gather_relu(AB, src_b, dst_b):
    @functools.partial(
        pl.kernel,
        out_type=jax.ShapeDtypeStruct((E2, HID), jnp.float32),
        mesh=_vmesh(),
        scratch_types=[
            pltpu.VMEM((CG,), jnp.int32),
            pltpu.VMEM((CG,), jnp.int32),
            pltpu.VMEM((CG, 2 * HID), jnp.float32),
            pltpu.VMEM((CG, 2 * HID), jnp.float32),
            pltpu.VMEM((CG, HID), jnp.float32),
            pltpu.SemaphoreType.DMA,
            pltpu.SemaphoreType.DMA,
        ],
    )
    def k(ab_hbm, src_hbm, dst_hbm, r_hbm, idxs_v, idxd_v, d_v, s_v, r_v,
          sem_a, sem_b):
        wid = lax.axis_index("s") * 2 + lax.axis_index("c")

        def chunk(i, _):
            base = wid * EPT + i * CG
            pltpu.sync_copy(src_hbm.at[pl.ds(base, CG)], idxs_v)
            pltpu.sync_copy(dst_hbm.at[pl.ds(base, CG)], idxd_v)
            ca = pltpu.async_copy(ab_hbm.at[idxd_v], d_v, sem_a)
            cb = pltpu.async_copy(ab_hbm.at[idxs_v], s_v, sem_b)
            ca.wait()
            cb.wait()

            def row(r, _):
                for q in range(HID // 16):
                    sl = pl.ds(16 * q, 16)
                    sh = pl.ds(HID + 16 * q, 16)
                    r_v[r, sl] = jnp.maximum(d_v[r, sl] + s_v[r, sh], 0.0)
                return 0

            lax.fori_loop(0, CG, row, 0)
            pltpu.sync_copy(r_v, r_hbm.at[pl.ds(base, CG)])
            return 0

        lax.fori_loop(0, EPT // CG, chunk, 0)

    return k(AB, src_b, dst_b)


def _sc_scatter_max(H, dstrel, offs, d):
    nq = d // 16

    @functools.partial(
        pl.kernel,
        out_type=jax.ShapeDtypeStruct((NPAD, d), jnp.float32),
        mesh=_vmesh(),
        scratch_types=[
            pltpu.VMEM((NB, 16), jnp.int32),
            pltpu.VMEM((CS + 16,), jnp.int32),
            pltpu.VMEM((CS, d), jnp.float32),
            pltpu.VMEM((NT, d), jnp.float32),
        ],
    )
    def k(h_hbm, dr_hbm, off_hbm, m_hbm, off_v, dr_v, h_v, acc_v):
        wid = lax.axis_index("s") * 2 + lax.axis_index("c")
        pltpu.sync_copy(off_hbm, off_v)
        neg = jnp.full((16,), -jnp.inf, jnp.float32)

        for half in range(2):
            b = wid * 2 + half
            ov = off_v[b, pl.ds(0, 16)]
            e0 = ov[0]
            e1 = ov[1]

            def initrow(r, _):
                for q in range(nq):
                    acc_v[r, pl.ds(16 * q, 16)] = neg
                return 0

            lax.fori_loop(0, NT, initrow, 0)

            a0 = (e0 // 8) * 8
            nch = (e1 - a0 + CS - 1) // CS

            def chunk(j, _):
                cb = a0 + j * CS
                pltpu.sync_copy(dr_hbm.at[pl.ds(cb, CS)],
                                dr_v.at[pl.ds(0, CS)])
                pltpu.sync_copy(h_hbm.at[pl.ds(cb, CS)], h_v)

                def group(kg, _):
                    kbase = kg * 16
                    dvec = dr_v[pl.ds(kbase, 16)]
                    for l in range(16):
                        g = cb + kbase + l
                        dd = dvec[l]
                        valid = jnp.logical_and(g >= e0, g < e1)
                        pad = jnp.where(valid, 0.0, -jnp.inf)
                        for q in range(nq):
                            sl = pl.ds(16 * q, 16)
                            hv = h_v[kbase + l, sl] + pad
                            acc_v[dd, sl] = jnp.maximum(acc_v[dd, sl], hv)
                    return 0

                lax.fori_loop(0, CS // 16, group, 0)
                return 0

            lax.fori_loop(0, nch, chunk, 0)
            pltpu.sync_copy(acc_v, m_hbm.at[pl.ds(b * NT, NT)])

    return k(H, dstrel, offs)


def _mm_kernel(r_ref, w_ref, o_ref):
    o_ref[...] = lax.dot_general(
        r_ref[...], w_ref[...], (((1,), (0,)), ((), ())),
        preferred_element_type=jnp.float32)


def _edge_matmul(r, w):
    d = w.shape[1]
    return pl.pallas_call(
        _mm_kernel,
        grid=(E2 // EBLK,),
        in_specs=[
            pl.BlockSpec((EBLK, HID), lambda i: (i, 0)),
            pl.BlockSpec((HID, d), lambda i: (0, 0)),
        ],
        out_specs=pl.BlockSpec((EBLK, d), lambda i: (i, 0)),
        out_shape=jax.ShapeDtypeStruct((E2, d), jnp.float32),
    )(r, w)


def _dense1_kernel(x_ref, w0_ref, b0_ref, wab_ref, bab_ref, ab_ref):
    f = jnp.maximum(
        lax.dot_general(x_ref[...], w0_ref[...], (((1,), (0,)), ((), ())),
                        preferred_element_type=jnp.float32) + b0_ref[...],
        0.0)
    ab_ref[...] = lax.dot_general(
        f, wab_ref[...], (((1,), (0,)), ((), ())),
        preferred_element_type=jnp.float32) + bab_ref[...]


def _dense1(x, w0t, b0, wabt, bab):
    nb = N // NUM_NODES
    return pl.pallas_call(
        _dense1_kernel,
        grid=(nb,),
        in_specs=[
            pl.BlockSpec((NUM_NODES, 2), lambda i: (i, 0)),
            pl.BlockSpec((2, HID), lambda i: (0, 0)),
            pl.BlockSpec((1, HID), lambda i: (0, 0)),
            pl.BlockSpec((HID, 2 * HID), lambda i: (0, 0)),
            pl.BlockSpec((1, 2 * HID), lambda i: (0, 0)),
        ],
        out_specs=pl.BlockSpec((NUM_NODES, 2 * HID), lambda i: (i, 0)),
        out_shape=jax.ShapeDtypeStruct((N, 2 * HID), jnp.float32),
    )(x, w0t, b0, wabt, bab)


def _dense23_kernel(m_ref, b2p_ref, wab_ref, bab_ref, sab_ref, ab_ref):
    m = m_ref[...]
    h = jnp.maximum(jnp.where(m > -3e38, m + b2p_ref[...], 0.0), 0.0)
    ab_ref[...] = (lax.dot_general(h, wab_ref[...], (((1,), (0,)), ((), ())),
                                   preferred_element_type=jnp.float32)
                   + sab_ref[0] + bab_ref[...])


def _dense23(m, b2p, wabt, bab, sab):
    nb = N // NUM_NODES
    return pl.pallas_call(
        _dense23_kernel,
        grid=(nb,),
        in_specs=[
            pl.BlockSpec((NUM_NODES, HID), lambda i: (i, 0)),
            pl.BlockSpec((1, HID), lambda i: (0, 0)),
            pl.BlockSpec((HID, 2 * HID), lambda i: (0, 0)),
            pl.BlockSpec((1, 2 * HID), lambda i: (0, 0)),
            pl.BlockSpec((1, 1, 2 * HID), lambda i: (i, 0, 0)),
        ],
        out_specs=pl.BlockSpec((NUM_NODES, 2 * HID), lambda i: (i, 0)),
        out_shape=jax.ShapeDtypeStruct((N, 2 * HID), jnp.float32),
    )(m, b2p, wabt, bab, sab[:, None, :])


def kernel(x, edge_index, batch, t, init_lin_W, init_lin_b, fourier_W,
           embed_W, embed_b, m1_W1, m1_b1, m1_W2, m1_b2, m2_W1, m2_b1,
           m2_W2, m2_b2, m3_W1, m3_b1, m3_W2, m3_b2):
    src = edge_index[0]
    dst = edge_index[1]

    order = jnp.argsort(dst)
    src_b = src[order]
    dst_b = dst[order]
    offs = jnp.searchsorted(
        dst_b, jnp.arange(0, NPAD + 1, NT, dtype=jnp.int32)).astype(jnp.int32)
    offs = jnp.stack([offs[:NB], offs[1:]], axis=1)
    offs = jnp.pad(offs, ((0, 0), (0, 14)))
    dstrel = dst_b % NT
    dstrel = jnp.zeros((E2,), jnp.int32).at[:E].set(dstrel)
    src_b = jnp.zeros((E2,), jnp.int32).at[:E].set(src_b)
    dst_b2 = jnp.zeros((E2,), jnp.int32).at[:E].set(dst_b)

    ts = t[:, 0]
    proj = ts[:, None] * fourier_W[None, :] * (2.0 * np.pi)
    four = jnp.concatenate([jnp.sin(proj), jnp.cos(proj)], axis=-1)
    emb = jax.nn.relu(four @ embed_W.T + embed_b)

    w1a1, w1b1 = m1_W1[:, :HID], m1_W1[:, HID:]
    wab1t = jnp.concatenate([(w1a1 - w1b1).T, w1b1.T], axis=1)
    bab1 = jnp.concatenate([m1_b1, jnp.zeros((HID,), jnp.float32)])[None, :]

    def split2(W1, b1v):
        din = HID + EMB
        w1a, w1b = W1[:, :din], W1[:, din:]
        wa = w1a - w1b
        wabt = jnp.concatenate([wa[:, :HID].T, w1b[:, :HID].T], axis=1)
        sab = jnp.concatenate([emb @ wa[:, HID:].T, emb @ w1b[:, HID:].T],
                              axis=1)
        bab = jnp.concatenate([b1v, jnp.zeros((HID,), jnp.float32)])[None, :]
        return wabt, bab, sab

    wab2t, bab2, sab2 = split2(m2_W1, m2_b1)
    wab3t, bab3, sab3 = split2(m3_W1, m3_b1)

    w2_1t = m1_W2.T
    w2_2t = m2_W2.T
    w2_3t = jnp.zeros((HID, 16), jnp.float32).at[:, :2].set(m3_W2.T)

    def _xla_scatter_max(h, d):
        agg = jax.ops.segment_max(h[:E], dst_b, num_segments=N)
        return jnp.zeros((NPAD, d), jnp.float32).at[:N].set(agg)

    ab1 = _dense1(x, init_lin_W.T, init_lin_b[None, :], wab1t, bab1)
    r1 = _sc_gather_relu(ab1, src_b, dst_b2)
    h1 = _edge_matmul(r1, w2_1t)
    m1 = _sc_scatter_max(h1, dstrel, offs, HID)

    ab2 = _dense23(m1[:N], m1_b2[None, :], wab2t, bab2, sab2)
    r2 = _sc_gather_relu(ab2, src_b, dst_b2)
    h2 = _edge_matmul(r2, w2_2t)
    m2 = _sc_scatter_max(h2, dstrel, offs, HID)

    ab3 = _dense23(m2[:N], m2_b2[None, :], wab3t, bab3, sab3)
    r3 = _sc_gather_relu(ab3, src_b, dst_b2)
    h3 = _edge_matmul(r3, w2_3t)
    m3 = _sc_scatter_max(h3, dstrel, offs, 16)

    agg = m3[:N, :2]
    out = jnp.where(agg > -3e38, agg + m3_b2, 0.0)
    std = jnp.sqrt((SIGMA ** (2.0 * jnp.repeat(ts, NUM_NODES)[:, None]) - 1.0)
                   / (2.0 * jnp.log(SIGMA)))
    return out / (std + 1e-07)


# speedup vs baseline: 2.3795x; 1.1944x over previous
import functools

import jax
import jax.numpy as jnp
import numpy as np
from jax import lax
from jax.experimental import pallas as pl
from jax.experimental.pallas import tpu as pltpu
from jax.experimental.pallas import tpu_sc as plsc

BS = 50
NUM_NODES = 1000
N = BS * NUM_NODES
E = 800000
HID = 64
EMB = 32
SIGMA = 25.0

NTILES = 32
NB = 64
NT = 800
NPAD = NB * NT
SUB = 128
CG = SUB
EPT = 196 * CG
E2 = NTILES * EPT
CS = 128
EBLK = 2048


def _vmesh():
    return plsc.VectorSubcoreMesh(core_axis_name="c", subcore_axis_name="s")


def _sc_gather_relu(A, B, src_b, dst_b):
    @functools.partial(
        pl.kernel,
        out_type=jax.ShapeDtypeStruct((E2, HID), jnp.float32),
        mesh=_vmesh(),
        compiler_params=pltpu.CompilerParams(use_tc_tiling_on_sc=False),
        scratch_types=[
            pltpu.VMEM((2, CG), jnp.int32),
            pltpu.VMEM((2, CG), jnp.int32),
            pltpu.VMEM((2, CG, HID), jnp.float32),
            pltpu.VMEM((2, CG, HID), jnp.float32),
            pltpu.VMEM((CG, HID), jnp.float32),
            pltpu.SemaphoreType.DMA,
            pltpu.SemaphoreType.DMA,
            pltpu.SemaphoreType.DMA,
            pltpu.SemaphoreType.DMA,
        ],
    )
    def k(a_hbm, b_hbm, src_hbm, dst_hbm, r_hbm, idxs_v, idxd_v, d_v, s_v,
          r_v, sem_d0, sem_d1, sem_s0, sem_s1):
        wid = lax.axis_index("s") * 2 + lax.axis_index("c")
        sem_d = (sem_d0, sem_d1)
        sem_s = (sem_s0, sem_s1)
        nch = EPT // CG

        def issue(ci, bb):
            base = wid * EPT + ci * CG
            pltpu.sync_copy(src_hbm.at[pl.ds(base, CG)], idxs_v.at[bb])
            pltpu.sync_copy(dst_hbm.at[pl.ds(base, CG)], idxd_v.at[bb])
            pltpu.async_copy(a_hbm.at[idxd_v.at[bb]], d_v.at[bb], sem_d[bb])
            pltpu.async_copy(b_hbm.at[idxs_v.at[bb]], s_v.at[bb], sem_s[bb])

        issue(0, 0)

        def pair(j, _):
            i0 = j * 2
            for bb in range(2):
                i = i0 + bb

                @pl.when(i + 1 < nch)
                def _():
                    issue(i + 1, 1 - bb)

                pltpu.make_async_copy(
                    a_hbm.at[idxd_v.at[bb]], d_v.at[bb], sem_d[bb]).wait()
                pltpu.make_async_copy(
                    b_hbm.at[idxs_v.at[bb]], s_v.at[bb], sem_s[bb]).wait()

                def row(r, _):
                    for q in range(HID // 16):
                        sl = pl.ds(16 * q, 16)
                        r_v[r, sl] = jnp.maximum(
                            d_v[bb, r, sl] + s_v[bb, r, sl], 0.0)
                    return 0

                lax.fori_loop(0, CG, row, 0)
                base = wid * EPT + i * CG
                pltpu.sync_copy(r_v, r_hbm.at[pl.ds(base, CG)])
            return 0

        lax.fori_loop(0, nch // 2, pair, 0)

    return k(A, B, src_b, dst_b)


def _sc_scatter_max(H, dstrel, offs, d):
    nq = d // 16

    @functools.partial(
        pl.kernel,
        out_type=jax.ShapeDtypeStruct((NPAD, d), jnp.float32),
        mesh=_vmesh(),
        scratch_types=[
            pltpu.VMEM((NB, 16), jnp.int32),
            pltpu.VMEM((CS + 16,), jnp.int32),
            pltpu.VMEM((CS, d), jnp.float32),
            pltpu.VMEM((NT, d), jnp.float32),
        ],
    )
    def k(h_hbm, dr_hbm, off_hbm, m_hbm, off_v, dr_v, h_v, acc_v):
        wid = lax.axis_index("s") * 2 + lax.axis_index("c")
        pltpu.sync_copy(off_hbm, off_v)
        neg = jnp.full((16,), -jnp.inf, jnp.float32)

        for half in range(2):
            b = wid * 2 + half
            ov = off_v[b, pl.ds(0, 16)]
            e0 = ov[0]
            e1 = ov[1]

            def initrow(r, _):
                for q in range(nq):
                    acc_v[r, pl.ds(16 * q, 16)] = neg
                return 0

            lax.fori_loop(0, NT, initrow, 0)

            a0 = (e0 // 8) * 8
            nch = (e1 - a0 + CS - 1) // CS

            def chunk(j, _):
                cb = a0 + j * CS
                pltpu.sync_copy(dr_hbm.at[pl.ds(cb, CS)],
                                dr_v.at[pl.ds(0, CS)])
                pltpu.sync_copy(h_hbm.at[pl.ds(cb, CS)], h_v)

                def group(kg, _):
                    kbase = kg * 16
                    dvec = dr_v[pl.ds(kbase, 16)]
                    for l in range(16):
                        g = cb + kbase + l
                        dd = dvec[l]
                        valid = jnp.logical_and(g >= e0, g < e1)
                        pad = jnp.where(valid, 0.0, -jnp.inf)
                        for q in range(nq):
                            sl = pl.ds(16 * q, 16)
                            hv = h_v[kbase + l, sl] + pad
                            acc_v[dd, sl] = jnp.maximum(acc_v[dd, sl], hv)
                    return 0

                lax.fori_loop(0, CS // 16, group, 0)
                return 0

            lax.fori_loop(0, nch, chunk, 0)
            pltpu.sync_copy(acc_v, m_hbm.at[pl.ds(b * NT, NT)])

    return k(H, dstrel, offs)


def _mm_kernel(r_ref, w_ref, o_ref):
    o_ref[...] = lax.dot_general(
        r_ref[...], w_ref[...], (((1,), (0,)), ((), ())),
        preferred_element_type=jnp.float32)


def _edge_matmul(r, w):
    d = w.shape[1]
    return pl.pallas_call(
        _mm_kernel,
        grid=(E2 // EBLK,),
        in_specs=[
            pl.BlockSpec((EBLK, HID), lambda i: (i, 0)),
            pl.BlockSpec((HID, d), lambda i: (0, 0)),
        ],
        out_specs=pl.BlockSpec((EBLK, d), lambda i: (i, 0)),
        out_shape=jax.ShapeDtypeStruct((E2, d), jnp.float32),
    )(r, w)


def _dense1_kernel(x_ref, w0_ref, b0_ref, wab_ref, bab_ref, a_ref, b_ref):
    f = jnp.maximum(
        lax.dot_general(x_ref[...], w0_ref[...], (((1,), (0,)), ((), ())),
                        preferred_element_type=jnp.float32) + b0_ref[...],
        0.0)
    ab = lax.dot_general(
        f, wab_ref[...], (((1,), (0,)), ((), ())),
        preferred_element_type=jnp.float32) + bab_ref[...]
    a_ref[...] = ab[:, :HID]
    b_ref[...] = ab[:, HID:]


def _dense1(x, w0t, b0, wabt, bab):
    nb = N // NUM_NODES
    return pl.pallas_call(
        _dense1_kernel,
        grid=(nb,),
        in_specs=[
            pl.BlockSpec((NUM_NODES, 2), lambda i: (i, 0)),
            pl.BlockSpec((2, HID), lambda i: (0, 0)),
            pl.BlockSpec((1, HID), lambda i: (0, 0)),
            pl.BlockSpec((HID, 2 * HID), lambda i: (0, 0)),
            pl.BlockSpec((1, 2 * HID), lambda i: (0, 0)),
        ],
        out_specs=[
            pl.BlockSpec((NUM_NODES, HID), lambda i: (i, 0)),
            pl.BlockSpec((NUM_NODES, HID), lambda i: (i, 0)),
        ],
        out_shape=[
            jax.ShapeDtypeStruct((N, HID), jnp.float32),
            jax.ShapeDtypeStruct((N, HID), jnp.float32),
        ],
    )(x, w0t, b0, wabt, bab)


def _dense23_kernel(m_ref, b2p_ref, wab_ref, bab_ref, sab_ref, a_ref, b_ref):
    m = m_ref[...]
    h = jnp.maximum(jnp.where(m > -3e38, m + b2p_ref[...], 0.0), 0.0)
    ab = (lax.dot_general(h, wab_ref[...], (((1,), (0,)), ((), ())),
                          preferred_element_type=jnp.float32)
          + sab_ref[0] + bab_ref[...])
    a_ref[...] = ab[:, :HID]
    b_ref[...] = ab[:, HID:]


def _dense23(m, b2p, wabt, bab, sab):
    nb = N // NUM_NODES
    return pl.pallas_call(
        _dense23_kernel,
        grid=(nb,),
        in_specs=[
            pl.BlockSpec((NUM_NODES, HID), lambda i: (i, 0)),
            pl.BlockSpec((1, HID), lambda i: (0, 0)),
            pl.BlockSpec((HID, 2 * HID), lambda i: (0, 0)),
            pl.BlockSpec((1, 2 * HID), lambda i: (0, 0)),
            pl.BlockSpec((1, 1, 2 * HID), lambda i: (i, 0, 0)),
        ],
        out_specs=[
            pl.BlockSpec((NUM_NODES, HID), lambda i: (i, 0)),
            pl.BlockSpec((NUM_NODES, HID), lambda i: (i, 0)),
        ],
        out_shape=[
            jax.ShapeDtypeStruct((N, HID), jnp.float32),
            jax.ShapeDtypeStruct((N, HID), jnp.float32),
        ],
    )(m, b2p, wabt, bab, sab[:, None, :])


def kernel(x, edge_index, batch, t, init_lin_W, init_lin_b, fourier_W,
           embed_W, embed_b, m1_W1, m1_b1, m1_W2, m1_b2, m2_W1, m2_b1,
           m2_W2, m2_b2, m3_W1, m3_b1, m3_W2, m3_b2):
    src = edge_index[0]
    dst = edge_index[1]

    order = jnp.argsort(dst)
    src_b = src[order]
    dst_b = dst[order]
    offs = jnp.searchsorted(
        dst_b, jnp.arange(0, NPAD + 1, NT, dtype=jnp.int32)).astype(jnp.int32)
    offs = jnp.stack([offs[:NB], offs[1:]], axis=1)
    offs = jnp.pad(offs, ((0, 0), (0, 14)))
    dstrel = dst_b % NT
    dstrel = jnp.zeros((E2,), jnp.int32).at[:E].set(dstrel)
    src_b = jnp.zeros((E2,), jnp.int32).at[:E].set(src_b)
    dst_b2 = jnp.zeros((E2,), jnp.int32).at[:E].set(dst_b)

    ts = t[:, 0]
    proj = ts[:, None] * fourier_W[None, :] * (2.0 * np.pi)
    four = jnp.concatenate([jnp.sin(proj), jnp.cos(proj)], axis=-1)
    emb = jax.nn.relu(four @ embed_W.T + embed_b)

    w1a1, w1b1 = m1_W1[:, :HID], m1_W1[:, HID:]
    wab1t = jnp.concatenate([(w1a1 - w1b1).T, w1b1.T], axis=1)
    bab1 = jnp.concatenate([m1_b1, jnp.zeros((HID,), jnp.float32)])[None, :]

    def split2(W1, b1v):
        din = HID + EMB
        w1a, w1b = W1[:, :din], W1[:, din:]
        wa = w1a - w1b
        wabt = jnp.concatenate([wa[:, :HID].T, w1b[:, :HID].T], axis=1)
        sab = jnp.concatenate([emb @ wa[:, HID:].T, emb @ w1b[:, HID:].T],
                              axis=1)
        bab = jnp.concatenate([b1v, jnp.zeros((HID,), jnp.float32)])[None, :]
        return wabt, bab, sab

    wab2t, bab2, sab2 = split2(m2_W1, m2_b1)
    wab3t, bab3, sab3 = split2(m3_W1, m3_b1)

    w2_1t = m1_W2.T
    w2_2t = m2_W2.T
    w2_3t = jnp.zeros((HID, 16), jnp.float32).at[:, :2].set(m3_W2.T)

    def _xla_scatter_max(h, d):
        agg = jax.ops.segment_max(h[:E], dst_b, num_segments=N)
        return jnp.zeros((NPAD, d), jnp.float32).at[:N].set(agg)

    a1, b1t_ = _dense1(x, init_lin_W.T, init_lin_b[None, :], wab1t, bab1)
    r1 = _sc_gather_relu(a1, b1t_, src_b, dst_b2)
    h1 = _edge_matmul(r1, w2_1t)
    m1 = _sc_scatter_max(h1, dstrel, offs, HID)

    a2, b2t_ = _dense23(m1[:N], m1_b2[None, :], wab2t, bab2, sab2)
    r2 = _sc_gather_relu(a2, b2t_, src_b, dst_b2)
    h2 = _edge_matmul(r2, w2_2t)
    m2 = _sc_scatter_max(h2, dstrel, offs, HID)

    a3, b3t_ = _dense23(m2[:N], m2_b2[None, :], wab3t, bab3, sab3)
    r3 = _sc_gather_relu(a3, b3t_, src_b, dst_b2)
    h3 = _edge_matmul(r3, w2_3t)
    m3 = _sc_scatter_max(h3, dstrel, offs, 16)

    agg = m3[:N, :2]
    out = jnp.where(agg > -3e38, agg + m3_b2, 0.0)
    std = jnp.sqrt((SIGMA ** (2.0 * jnp.repeat(ts, NUM_NODES)[:, None]) - 1.0)
                   / (2.0 * jnp.log(SIGMA)))
    return out / (std + 1e-07)


# speedup vs baseline: 2.3923x; 1.0054x over previous
import functools

import jax
import jax.numpy as jnp
import numpy as np
from jax import lax
from jax.experimental import pallas as pl
from jax.experimental.pallas import tpu as pltpu
from jax.experimental.pallas import tpu_sc as plsc

BS = 50
NUM_NODES = 1000
N = BS * NUM_NODES
E = 800000
HID = 64
EMB = 32
SIGMA = 25.0

NTILES = 32
NB = 64
NT = 800
NPAD = NB * NT
SUB = 128
CG = SUB
EPT = 196 * CG
E2 = NTILES * EPT
CS = 128
EBLK = 2048


def _vmesh():
    return plsc.VectorSubcoreMesh(core_axis_name="c", subcore_axis_name="s")


def _sc_gather_relu(A, B, src_b, dst_b):
    @functools.partial(
        pl.kernel,
        out_type=jax.ShapeDtypeStruct((E2, HID), jnp.float32),
        mesh=_vmesh(),
        compiler_params=pltpu.CompilerParams(use_tc_tiling_on_sc=False),
        scratch_types=[
            pltpu.VMEM((2, CG), jnp.int32),
            pltpu.VMEM((2, CG), jnp.int32),
            pltpu.VMEM((2, CG, HID), jnp.float32),
            pltpu.VMEM((2, CG, HID), jnp.float32),
            pltpu.VMEM((CG, HID), jnp.float32),
            pltpu.SemaphoreType.DMA,
            pltpu.SemaphoreType.DMA,
            pltpu.SemaphoreType.DMA,
            pltpu.SemaphoreType.DMA,
        ],
    )
    def k(a_hbm, b_hbm, src_hbm, dst_hbm, r_hbm, idxs_v, idxd_v, d_v, s_v,
          r_v, sem_d0, sem_d1, sem_s0, sem_s1):
        wid = lax.axis_index("s") * 2 + lax.axis_index("c")
        sem_d = (sem_d0, sem_d1)
        sem_s = (sem_s0, sem_s1)
        nch = EPT // CG

        def issue(ci, bb):
            base = wid * EPT + ci * CG
            pltpu.sync_copy(src_hbm.at[pl.ds(base, CG)], idxs_v.at[bb])
            pltpu.sync_copy(dst_hbm.at[pl.ds(base, CG)], idxd_v.at[bb])
            pltpu.async_copy(a_hbm.at[idxd_v.at[bb]], d_v.at[bb], sem_d[bb])
            pltpu.async_copy(b_hbm.at[idxs_v.at[bb]], s_v.at[bb], sem_s[bb])

        issue(0, 0)

        def pair(j, _):
            i0 = j * 2
            for bb in range(2):
                i = i0 + bb

                @pl.when(i + 1 < nch)
                def _():
                    issue(i + 1, 1 - bb)

                pltpu.make_async_copy(
                    a_hbm.at[idxd_v.at[bb]], d_v.at[bb], sem_d[bb]).wait()
                pltpu.make_async_copy(
                    b_hbm.at[idxs_v.at[bb]], s_v.at[bb], sem_s[bb]).wait()

                def row(r, _):
                    for q in range(HID // 16):
                        sl = pl.ds(16 * q, 16)
                        r_v[r, sl] = jnp.maximum(
                            d_v[bb, r, sl] + s_v[bb, r, sl], 0.0)
                    return 0

                lax.fori_loop(0, CG, row, 0)
                base = wid * EPT + i * CG
                pltpu.sync_copy(r_v, r_hbm.at[pl.ds(base, CG)])
            return 0

        lax.fori_loop(0, nch // 2, pair, 0)

    return k(A, B, src_b, dst_b)


def _sc_scatter_max(H, dstrel, offs, d):
    nq = d // 16

    @functools.partial(
        pl.kernel,
        out_type=jax.ShapeDtypeStruct((NPAD, d), jnp.float32),
        mesh=_vmesh(),
        scratch_types=[
            pltpu.VMEM((NB, 16), jnp.int32),
            pltpu.VMEM((CS + 16,), jnp.int32),
            pltpu.VMEM((CS, d), jnp.float32),
            pltpu.VMEM((NT, d), jnp.float32),
        ],
    )
    def k(h_hbm, dr_hbm, off_hbm, m_hbm, off_v, dr_v, h_v, acc_v):
        wid = lax.axis_index("s") * 2 + lax.axis_index("c")
        pltpu.sync_copy(off_hbm, off_v)
        neg = jnp.full((16,), -jnp.inf, jnp.float32)

        for half in range(2):
            b = wid * 2 + half
            ov = off_v[b, pl.ds(0, 16)]
            e0 = ov[0]
            e1 = ov[1]

            def initrow(r, _):
                for q in range(nq):
                    acc_v[r, pl.ds(16 * q, 16)] = neg
                return 0

            lax.fori_loop(0, NT, initrow, 0)

            a0 = (e0 // 8) * 8
            nch = (e1 - a0 + CS - 1) // CS

            def chunk(j, _):
                cb = a0 + j * CS
                pltpu.sync_copy(dr_hbm.at[pl.ds(cb, CS)],
                                dr_v.at[pl.ds(0, CS)])
                pltpu.sync_copy(h_hbm.at[pl.ds(cb, CS)], h_v)

                def group(kg, _):
                    kbase = kg * 16
                    dvec = dr_v[pl.ds(kbase, 16)]
                    for l in range(16):
                        g = cb + kbase + l
                        dd = dvec[l]
                        valid = jnp.logical_and(g >= e0, g < e1)
                        pad = jnp.where(valid, 0.0, -jnp.inf)
                        for q in range(nq):
                            sl = pl.ds(16 * q, 16)
                            hv = h_v[kbase + l, sl] + pad
                            acc_v[dd, sl] = jnp.maximum(acc_v[dd, sl], hv)
                    return 0

                lax.fori_loop(0, CS // 16, group, 0)
                return 0

            lax.fori_loop(0, nch, chunk, 0)
            pltpu.sync_copy(acc_v, m_hbm.at[pl.ds(b * NT, NT)])

    return k(H, dstrel, offs)


def _mm_kernel(r_ref, w_ref, o_ref):
    o_ref[...] = lax.dot_general(
        r_ref[...], w_ref[...], (((1,), (0,)), ((), ())),
        preferred_element_type=jnp.float32)


def _edge_matmul(r, w):
    d = w.shape[1]
    return pl.pallas_call(
        _mm_kernel,
        grid=(E2 // EBLK,),
        in_specs=[
            pl.BlockSpec((EBLK, HID), lambda i: (i, 0)),
            pl.BlockSpec((HID, d), lambda i: (0, 0)),
        ],
        out_specs=pl.BlockSpec((EBLK, d), lambda i: (i, 0)),
        out_shape=jax.ShapeDtypeStruct((E2, d), jnp.float32),
    )(r, w)


def _dense1_kernel(x_ref, w0_ref, b0_ref, wab_ref, bab_ref, a_ref, b_ref):
    f = jnp.maximum(
        lax.dot_general(x_ref[...], w0_ref[...], (((1,), (0,)), ((), ())),
                        preferred_element_type=jnp.float32) + b0_ref[...],
        0.0)
    ab = lax.dot_general(
        f, wab_ref[...], (((1,), (0,)), ((), ())),
        preferred_element_type=jnp.float32) + bab_ref[...]
    a_ref[...] = ab[:, :HID]
    b_ref[...] = ab[:, HID:]


def _dense1(x, w0t, b0, wabt, bab):
    nb = N // NUM_NODES
    return pl.pallas_call(
        _dense1_kernel,
        grid=(nb,),
        in_specs=[
            pl.BlockSpec((NUM_NODES, 2), lambda i: (i, 0)),
            pl.BlockSpec((2, HID), lambda i: (0, 0)),
            pl.BlockSpec((1, HID), lambda i: (0, 0)),
            pl.BlockSpec((HID, 2 * HID), lambda i: (0, 0)),
            pl.BlockSpec((1, 2 * HID), lambda i: (0, 0)),
        ],
        out_specs=[
            pl.BlockSpec((NUM_NODES, HID), lambda i: (i, 0)),
            pl.BlockSpec((NUM_NODES, HID), lambda i: (i, 0)),
        ],
        out_shape=[
            jax.ShapeDtypeStruct((N, HID), jnp.float32),
            jax.ShapeDtypeStruct((N, HID), jnp.float32),
        ],
    )(x, w0t, b0, wabt, bab)


def _dense23_kernel(m_ref, b2p_ref, wab_ref, bab_ref, sab_ref, a_ref, b_ref):
    m = m_ref[...]
    h = jnp.maximum(jnp.where(m > -3e38, m + b2p_ref[...], 0.0), 0.0)
    ab = (lax.dot_general(h, wab_ref[...], (((1,), (0,)), ((), ())),
                          preferred_element_type=jnp.float32)
          + sab_ref[0] + bab_ref[...])
    a_ref[...] = ab[:, :HID]
    b_ref[...] = ab[:, HID:]


def _dense23(m, b2p, wabt, bab, sab):
    nb = N // NUM_NODES
    return pl.pallas_call(
        _dense23_kernel,
        grid=(nb,),
        in_specs=[
            pl.BlockSpec((NUM_NODES, HID), lambda i: (i, 0)),
            pl.BlockSpec((1, HID), lambda i: (0, 0)),
            pl.BlockSpec((HID, 2 * HID), lambda i: (0, 0)),
            pl.BlockSpec((1, 2 * HID), lambda i: (0, 0)),
            pl.BlockSpec((1, 1, 2 * HID), lambda i: (i, 0, 0)),
        ],
        out_specs=[
            pl.BlockSpec((NUM_NODES, HID), lambda i: (i, 0)),
            pl.BlockSpec((NUM_NODES, HID), lambda i: (i, 0)),
        ],
        out_shape=[
            jax.ShapeDtypeStruct((N, HID), jnp.float32),
            jax.ShapeDtypeStruct((N, HID), jnp.float32),
        ],
    )(m, b2p, wabt, bab, sab[:, None, :])


def kernel(x, edge_index, batch, t, init_lin_W, init_lin_b, fourier_W,
           embed_W, embed_b, m1_W1, m1_b1, m1_W2, m1_b2, m2_W1, m2_b1,
           m2_W2, m2_b2, m3_W1, m3_b1, m3_W2, m3_b2):
    src = edge_index[0]
    dst = edge_index[1]

    dst_b, src_b = lax.sort((dst, src), num_keys=1)
    offs = jnp.searchsorted(
        dst_b, jnp.arange(0, NPAD + 1, NT, dtype=jnp.int32)).astype(jnp.int32)
    offs = jnp.stack([offs[:NB], offs[1:]], axis=1)
    offs = jnp.pad(offs, ((0, 0), (0, 14)))
    dstrel = dst_b % NT
    dstrel = jnp.zeros((E2,), jnp.int32).at[:E].set(dstrel)
    src_b = jnp.zeros((E2,), jnp.int32).at[:E].set(src_b)
    dst_b2 = jnp.zeros((E2,), jnp.int32).at[:E].set(dst_b)

    ts = t[:, 0]
    proj = ts[:, None] * fourier_W[None, :] * (2.0 * np.pi)
    four = jnp.concatenate([jnp.sin(proj), jnp.cos(proj)], axis=-1)
    emb = jax.nn.relu(four @ embed_W.T + embed_b)

    w1a1, w1b1 = m1_W1[:, :HID], m1_W1[:, HID:]
    wab1t = jnp.concatenate([(w1a1 - w1b1).T, w1b1.T], axis=1)
    bab1 = jnp.concatenate([m1_b1, jnp.zeros((HID,), jnp.float32)])[None, :]

    def split2(W1, b1v):
        din = HID + EMB
        w1a, w1b = W1[:, :din], W1[:, din:]
        wa = w1a - w1b
        wabt = jnp.concatenate([wa[:, :HID].T, w1b[:, :HID].T], axis=1)
        sab = jnp.concatenate([emb @ wa[:, HID:].T, emb @ w1b[:, HID:].T],
                              axis=1)
        bab = jnp.concatenate([b1v, jnp.zeros((HID,), jnp.float32)])[None, :]
        return wabt, bab, sab

    wab2t, bab2, sab2 = split2(m2_W1, m2_b1)
    wab3t, bab3, sab3 = split2(m3_W1, m3_b1)

    w2_1t = m1_W2.T
    w2_2t = m2_W2.T
    w2_3t = jnp.zeros((HID, 16), jnp.float32).at[:, :2].set(m3_W2.T)

    def _xla_scatter_max(h, d):
        agg = jax.ops.segment_max(h[:E], dst_b, num_segments=N)
        return jnp.zeros((NPAD, d), jnp.float32).at[:N].set(agg)

    a1, b1t_ = _dense1(x, init_lin_W.T, init_lin_b[None, :], wab1t, bab1)
    r1 = _sc_gather_relu(a1, b1t_, src_b, dst_b2)
    h1 = _edge_matmul(r1, w2_1t)
    m1 = _sc_scatter_max(h1, dstrel, offs, HID)

    a2, b2t_ = _dense23(m1[:N], m1_b2[None, :], wab2t, bab2, sab2)
    r2 = _sc_gather_relu(a2, b2t_, src_b, dst_b2)
    h2 = _edge_matmul(r2, w2_2t)
    m2 = _sc_scatter_max(h2, dstrel, offs, HID)

    a3, b3t_ = _dense23(m2[:N], m2_b2[None, :], wab3t, bab3, sab3)
    r3 = _sc_gather_relu(a3, b3t_, src_b, dst_b2)
    h3 = _edge_matmul(r3, w2_3t)
    m3 = _sc_scatter_max(h3, dstrel, offs, 16)

    agg = m3[:N, :2]
    out = jnp.where(agg > -3e38, agg + m3_b2, 0.0)
    std = jnp.sqrt((SIGMA ** (2.0 * jnp.repeat(ts, NUM_NODES)[:, None]) - 1.0)
                   / (2.0 * jnp.log(SIGMA)))
    return out / (std + 1e-07)
